# Initial kernel scaffold; baseline (speedup 1.0000x reference)
import jax
import jax.numpy as jnp
from jax import lax
from jax.experimental import pallas as pl
from jax.experimental.pallas import tpu as pltpu
from jax.experimental.pallas import tpu_sc as plsc

N = 10000
E = 320000
H = 8
VD = 128
HD = VD // H

ROWS = E // 128
CHUNK_ROWS = 2
NCHUNKS = ROWS // CHUNK_ROWS
NWORKERS = 32
ROWS_PER_SUB = N // 16
EPS = 1e-16


_B1 = 2560


def _stage1_body(cut_ref, ew_ref, v_ref, wv_ref, pden_ref):
    p = jnp.exp(cut_ref[...] * ew_ref[...])
    b = p.shape[0]
    pden_ref[...] = jnp.concatenate(
        [p, jnp.zeros((b, H), jnp.float32)], axis=1)
    pfull = jnp.concatenate(
        [jnp.broadcast_to(p[:, h:h + 1], (b, HD)) for h in range(H)], axis=1)
    wv_ref[...] = v_ref[...] * pfull


def _stage1(cut, ew, value):
    grid = (E // _B1,)
    return pl.pallas_call(
        _stage1_body,
        grid=grid,
        in_specs=[
            pl.BlockSpec((_B1, 1), lambda i: (i, 0)),
            pl.BlockSpec((_B1, H), lambda i: (i, 0)),
            pl.BlockSpec((_B1, VD), lambda i: (i, 0)),
        ],
        out_specs=[
            pl.BlockSpec((_B1, VD), lambda i: (i, 0)),
            pl.BlockSpec((_B1, HD), lambda i: (i, 0)),
        ],
        out_shape=[
            jax.ShapeDtypeStruct((E, VD), jnp.float32),
            jax.ShapeDtypeStruct((E, HD), jnp.float32),
        ],
    )(cut, ew, value)


_sc_mesh = plsc.VectorSubcoreMesh(core_axis_name="c", subcore_axis_name="s")


def _stage2_body(wv_hbm, pden_hbm, dst_hbm, zn_hbm, zd_hbm,
                 num_hbm, den_hbm,
                 wv_buf, pden_buf, idx_buf, num_sh, den_sh):
    cid = lax.axis_index("c")
    sid = lax.axis_index("s")
    wid = cid * 16 + sid

    row0 = sid * ROWS_PER_SUB
    pltpu.sync_copy(zn_hbm.at[pl.ds(row0, ROWS_PER_SUB)],
                    num_sh.at[pl.ds(row0, ROWS_PER_SUB)])
    pltpu.sync_copy(zd_hbm.at[pl.ds(row0, ROWS_PER_SUB)],
                    den_sh.at[pl.ds(row0, ROWS_PER_SUB)])
    plsc.subcore_barrier()

    @pl.loop(wid, NCHUNKS, step=NWORKERS)
    def _(k):
        r = k * CHUNK_ROWS
        e = r * 128
        pltpu.sync_copy(wv_hbm.at[pl.ds(e, CHUNK_ROWS * 128)], wv_buf)
        pltpu.sync_copy(pden_hbm.at[pl.ds(e, CHUNK_ROWS * 128)], pden_buf)
        pltpu.sync_copy(dst_hbm.at[pl.ds(r, CHUNK_ROWS)], idx_buf)
        for j in range(CHUNK_ROWS):
            pltpu.sync_copy(wv_buf.at[pl.ds(j * 128, 128)],
                            num_sh.at[idx_buf.at[j]], add=True)
            pltpu.sync_copy(pden_buf.at[pl.ds(j * 128, 128)],
                            den_sh.at[idx_buf.at[j]], add=True)

    plsc.subcore_barrier()

    pltpu.sync_copy(num_sh.at[pl.ds(row0, ROWS_PER_SUB)],
                    num_hbm.at[cid, pl.ds(row0, ROWS_PER_SUB)])
    pltpu.sync_copy(den_sh.at[pl.ds(row0, ROWS_PER_SUB)],
                    den_hbm.at[cid, pl.ds(row0, ROWS_PER_SUB)])


def _stage2(wv, pden, dst_rows, zn, zd):
    kern = pl.kernel(
        _stage2_body,
        out_type=(jax.ShapeDtypeStruct((2, N, VD), jnp.float32),
                  jax.ShapeDtypeStruct((2, N, HD), jnp.float32)),
        mesh=_sc_mesh,
        scratch_types=[
            pltpu.VMEM((CHUNK_ROWS * 128, VD), jnp.float32),
            pltpu.VMEM((CHUNK_ROWS * 128, HD), jnp.float32),
            pltpu.VMEM((CHUNK_ROWS, 128), jnp.int32),
            pltpu.VMEM_SHARED((N, VD), jnp.float32),
            pltpu.VMEM_SHARED((N, HD), jnp.float32),
        ],
    )
    return kern(wv, pden, dst_rows, zn, zd)


_B3 = 2000


def _stage3_body(num_ref, den_ref, out_ref):
    nm = num_ref[0] + num_ref[1]
    dn = den_ref[0] + den_ref[1]
    b = nm.shape[0]
    dfull = jnp.concatenate(
        [jnp.broadcast_to(dn[:, h:h + 1], (b, HD)) for h in range(H)], axis=1)
    out_ref[...] = nm / (dfull + EPS)


def _stage3(num, den):
    grid = (N // _B3,)
    return pl.pallas_call(
        _stage3_body,
        grid=grid,
        in_specs=[
            pl.BlockSpec((2, _B3, VD), lambda i: (0, i, 0)),
            pl.BlockSpec((2, _B3, HD), lambda i: (0, i, 0)),
        ],
        out_specs=pl.BlockSpec((_B3, VD), lambda i: (i, 0)),
        out_shape=jax.ShapeDtypeStruct((N, VD), jnp.float32),
    )(num, den)


@jax.jit
def kernel(value, edge_weights, edge_weights_cutoff, edge_index):
    dst = edge_index[1].astype(jnp.int32).reshape(ROWS, 128)
    cut = edge_weights_cutoff.reshape(E, 1)
    wv, pden = _stage1(cut, edge_weights, value)
    zn = jnp.zeros((N, VD), jnp.float32)
    zd = jnp.zeros((N, HD), jnp.float32)
    num, den = _stage2(wv, pden, dst, zn, zd)
    return _stage3(num, den)


# speedup vs baseline: 34.2001x; 34.2001x over previous
import jax
import jax.numpy as jnp
from jax import lax
from jax.experimental import pallas as pl
from jax.experimental.pallas import tpu as pltpu
from jax.experimental.pallas import tpu_sc as plsc

N = 10000
E = 320000
H = 8
VD = 128
HD = VD // H

ROWS = E // 128
GROUP = 4
NGROUPS = ROWS // GROUP
NWORKERS = 32
GSTEPS = -(-NGROUPS // NWORKERS)
NPAD = 10240
ROWS_PER_SUB = NPAD // 16
EPS = 1e-16


_B1 = 2560


def _stage1_body(cut_ref, ew_ref, v_ref, wv_ref, pden_ref):
    p = jnp.exp(cut_ref[...] * ew_ref[...])
    b = p.shape[0]
    pden_ref[...] = jnp.concatenate(
        [p, jnp.zeros((b, H), jnp.float32)], axis=1)
    pfull = jnp.concatenate(
        [jnp.broadcast_to(p[:, h:h + 1], (b, HD)) for h in range(H)], axis=1)
    wv_ref[...] = v_ref[...] * pfull


def _stage1(cut, ew, value):
    grid = (E // _B1,)
    return pl.pallas_call(
        _stage1_body,
        grid=grid,
        in_specs=[
            pl.BlockSpec((_B1, 1), lambda i: (i, 0)),
            pl.BlockSpec((_B1, H), lambda i: (i, 0)),
            pl.BlockSpec((_B1, VD), lambda i: (i, 0)),
        ],
        out_specs=[
            pl.BlockSpec((_B1, VD), lambda i: (i, 0)),
            pl.BlockSpec((_B1, HD), lambda i: (i, 0)),
        ],
        out_shape=[
            jax.ShapeDtypeStruct((E, VD), jnp.float32),
            jax.ShapeDtypeStruct((E, HD), jnp.float32),
        ],
    )(cut, ew, value)


_sc_mesh = plsc.VectorSubcoreMesh(core_axis_name="c", subcore_axis_name="s")


def _num_body(wv_hbm, dst_hbm, z_hbm, acc_hbm,
              buf, idx0, idx1, idx2, idx3, acc_sh):
    cid = lax.axis_index("c")
    sid = lax.axis_index("s")
    wid = cid * 16 + sid

    row0 = sid * ROWS_PER_SUB
    pltpu.sync_copy(z_hbm, buf)
    for t in range(ROWS_PER_SUB // 128):
        pltpu.sync_copy(buf, acc_sh.at[pl.ds(row0 + t * 128, 128)])
    plsc.subcore_barrier()

    @pl.loop(0, GSTEPS)
    def _(i):
        g = wid + i * NWORKERS

        @pl.when(g < NGROUPS)
        def _():
            for j, idxj in enumerate((idx0, idx1, idx2, idx3)):
                e = (g * GROUP + j) * 128
                pltpu.sync_copy(dst_hbm.at[g, j], idxj)
                pltpu.sync_copy(wv_hbm.at[pl.ds(e, 128)], buf)
                pltpu.sync_copy(buf, acc_sh.at[idxj], add=True)

    plsc.subcore_barrier()

    for t in range(ROWS_PER_SUB // 128):
        r = row0 + t * 128
        pltpu.sync_copy(acc_sh.at[pl.ds(r, 128)], buf)
        pltpu.sync_copy(buf, acc_hbm.at[cid, pl.ds(r, 128)])


def _den_body(pden_hbm, dst_hbm, z_hbm, acc_hbm,
              buf, pbuf, idx0, idx1, idx2, idx3, acc_sh):
    cid = lax.axis_index("c")
    sid = lax.axis_index("s")
    wid = cid * 16 + sid

    row0 = sid * ROWS_PER_SUB
    pltpu.sync_copy(z_hbm, buf)
    for t in range(ROWS_PER_SUB // 128):
        pltpu.sync_copy(buf, acc_sh.at[pl.ds(row0 + t * 128, 128)])
    plsc.subcore_barrier()

    @pl.loop(0, GSTEPS)
    def _(i):
        g = wid + i * NWORKERS

        @pl.when(g < NGROUPS)
        def _():
            for j, idxj in enumerate((idx0, idx1, idx2, idx3)):
                e = (g * GROUP + j) * 128
                pltpu.sync_copy(dst_hbm.at[g, j], idxj)
                pltpu.sync_copy(pden_hbm.at[pl.ds(e, 128)], pbuf)

                @pl.loop(0, 128)
                def _(r):
                    buf[r, pl.ds(0, HD)] = pbuf[r]

                pltpu.sync_copy(buf, acc_sh.at[idxj], add=True)

    plsc.subcore_barrier()

    for t in range(ROWS_PER_SUB // 128):
        r = row0 + t * 128
        pltpu.sync_copy(acc_sh.at[pl.ds(r, 128)], buf)
        pltpu.sync_copy(buf, acc_hbm.at[cid, pl.ds(r, 128)])


def _scatter_pass(body, data, dst_rows, z, extra_scratch):
    kern = pl.kernel(
        body,
        out_type=jax.ShapeDtypeStruct((2, NPAD, VD), jnp.float32),
        mesh=_sc_mesh,
        scratch_types=[pltpu.VMEM((128, VD), jnp.float32)] + extra_scratch + [
            pltpu.VMEM((128,), jnp.int32),
            pltpu.VMEM((128,), jnp.int32),
            pltpu.VMEM((128,), jnp.int32),
            pltpu.VMEM((128,), jnp.int32),
            pltpu.VMEM_SHARED((NPAD, VD), jnp.float32),
        ],
    )
    return kern(data, dst_rows, z)


_B3 = 2000


def _stage4_body(num_ref, den_ref, out_ref):
    nm = num_ref[0] + num_ref[1]
    dn = den_ref[0] + den_ref[1]
    b = nm.shape[0]
    dfull = jnp.concatenate(
        [jnp.broadcast_to(dn[:, h:h + 1], (b, HD)) for h in range(H)], axis=1)
    out_ref[...] = nm / (dfull + EPS)


def _stage4(num, den):
    grid = (N // _B3,)
    return pl.pallas_call(
        _stage4_body,
        grid=grid,
        in_specs=[
            pl.BlockSpec((2, _B3, VD), lambda i: (0, i, 0)),
            pl.BlockSpec((2, _B3, VD), lambda i: (0, i, 0)),
        ],
        out_specs=pl.BlockSpec((_B3, VD), lambda i: (i, 0)),
        out_shape=jax.ShapeDtypeStruct((N, VD), jnp.float32),
    )(num, den)


@jax.jit
def kernel(value, edge_weights, edge_weights_cutoff, edge_index):
    dst = edge_index[1].astype(jnp.int32).reshape(NGROUPS, GROUP, 128)
    cut = edge_weights_cutoff.reshape(E, 1)
    wv, pden = _stage1(cut, edge_weights, value)
    z = jnp.zeros((128, VD), jnp.float32)
    num = _scatter_pass(_num_body, wv, dst, z, [])
    den = _scatter_pass(_den_body, pden, dst, z,
                        [pltpu.VMEM((128, HD), jnp.float32)])
    return _stage4(num, den)


# speedup vs baseline: 37.4267x; 1.0943x over previous
import jax
import jax.numpy as jnp
from jax import lax
from jax.experimental import pallas as pl
from jax.experimental.pallas import tpu as pltpu
from jax.experimental.pallas import tpu_sc as plsc

N = 10000
E = 320000
H = 8
VD = 128
HD = VD // H

ROWS = E // 128
GROUP = 4
NGROUPS = ROWS // GROUP
NWORKERS = 32
GSTEPS = -(-NGROUPS // NWORKERS)
NPAD = 10240
ROWS_PER_SUB = NPAD // 16
EPS = 1e-16


_B1 = 2560


def _stage1_body(w_ref, v_ref, wv_ref, pden_ref):
    p = jnp.exp(w_ref[...])
    b = p.shape[0]
    pden_ref[...] = jnp.concatenate(
        [p, jnp.zeros((b, H), jnp.float32)], axis=1)
    pfull = jnp.concatenate(
        [jnp.broadcast_to(p[:, h:h + 1], (b, HD)) for h in range(H)], axis=1)
    wv_ref[...] = v_ref[...] * pfull


def _stage1(w, value):
    grid = (E // _B1,)
    return pl.pallas_call(
        _stage1_body,
        grid=grid,
        in_specs=[
            pl.BlockSpec((_B1, H), lambda i: (i, 0)),
            pl.BlockSpec((_B1, VD), lambda i: (i, 0)),
        ],
        out_specs=[
            pl.BlockSpec((_B1, VD), lambda i: (i, 0)),
            pl.BlockSpec((_B1, HD), lambda i: (i, 0)),
        ],
        out_shape=[
            jax.ShapeDtypeStruct((E, VD), jnp.float32),
            jax.ShapeDtypeStruct((E, HD), jnp.float32),
        ],
    )(w, value)


_sc_mesh = plsc.VectorSubcoreMesh(core_axis_name="c", subcore_axis_name="s")


def _num_body(wv_hbm, dst_hbm, z_hbm, acc_hbm,
              buf, idx0, idx1, idx2, idx3, acc_sh):
    cid = lax.axis_index("c")
    sid = lax.axis_index("s")
    wid = cid * 16 + sid

    row0 = sid * ROWS_PER_SUB
    pltpu.sync_copy(z_hbm, buf)
    for t in range(ROWS_PER_SUB // 128):
        pltpu.sync_copy(buf, acc_sh.at[pl.ds(row0 + t * 128, 128)])
    plsc.subcore_barrier()

    @pl.loop(0, GSTEPS)
    def _(i):
        g = wid + i * NWORKERS

        @pl.when(g < NGROUPS)
        def _():
            for j, idxj in enumerate((idx0, idx1, idx2, idx3)):
                e = (g * GROUP + j) * 128
                pltpu.sync_copy(dst_hbm.at[pl.ds(e, 128)], idxj)
                pltpu.sync_copy(wv_hbm.at[pl.ds(e, 128)], buf)
                pltpu.sync_copy(buf, acc_sh.at[idxj], add=True)

    plsc.subcore_barrier()

    for t in range(ROWS_PER_SUB // 128):
        r = row0 + t * 128
        pltpu.sync_copy(acc_sh.at[pl.ds(r, 128)], buf)
        pltpu.sync_copy(buf, acc_hbm.at[cid, pl.ds(r, 128)])


def _den_body(pden_hbm, dst_hbm, z_hbm, acc_hbm,
              buf, pbuf, idx0, idx1, idx2, idx3, acc_sh):
    cid = lax.axis_index("c")
    sid = lax.axis_index("s")
    wid = cid * 16 + sid

    row0 = sid * ROWS_PER_SUB
    pltpu.sync_copy(z_hbm, buf)
    for t in range(ROWS_PER_SUB // 128):
        pltpu.sync_copy(buf, acc_sh.at[pl.ds(row0 + t * 128, 128)])
    plsc.subcore_barrier()

    @pl.loop(0, GSTEPS)
    def _(i):
        g = wid + i * NWORKERS

        @pl.when(g < NGROUPS)
        def _():
            for j, idxj in enumerate((idx0, idx1, idx2, idx3)):
                e = (g * GROUP + j) * 128
                pltpu.sync_copy(dst_hbm.at[pl.ds(e, 128)], idxj)
                pltpu.sync_copy(pden_hbm.at[pl.ds(e, 128)], pbuf)

                @pl.loop(0, 128)
                def _(r):
                    buf[r, pl.ds(0, HD)] = pbuf[r]

                pltpu.sync_copy(buf, acc_sh.at[idxj], add=True)

    plsc.subcore_barrier()

    for t in range(ROWS_PER_SUB // 128):
        r = row0 + t * 128
        pltpu.sync_copy(acc_sh.at[pl.ds(r, 128)], buf)
        pltpu.sync_copy(buf, acc_hbm.at[cid, pl.ds(r, 128)])


def _scatter_pass(body, data, dst_rows, z, extra_scratch):
    kern = pl.kernel(
        body,
        out_type=jax.ShapeDtypeStruct((2, NPAD, VD), jnp.float32),
        mesh=_sc_mesh,
        scratch_types=[pltpu.VMEM((128, VD), jnp.float32)] + extra_scratch + [
            pltpu.VMEM((128,), jnp.int32),
            pltpu.VMEM((128,), jnp.int32),
            pltpu.VMEM((128,), jnp.int32),
            pltpu.VMEM((128,), jnp.int32),
            pltpu.VMEM_SHARED((NPAD, VD), jnp.float32),
        ],
    )
    return kern(data, dst_rows, z)


_B3 = 2000


def _stage4_body(num_ref, den_ref, out_ref):
    nm = num_ref[0] + num_ref[1]
    dn = den_ref[0] + den_ref[1]
    b = nm.shape[0]
    dfull = jnp.concatenate(
        [jnp.broadcast_to(dn[:, h:h + 1], (b, HD)) for h in range(H)], axis=1)
    out_ref[...] = nm / (dfull + EPS)


def _stage4(num, den):
    grid = (N // _B3,)
    return pl.pallas_call(
        _stage4_body,
        grid=grid,
        in_specs=[
            pl.BlockSpec((2, _B3, VD), lambda i: (0, i, 0)),
            pl.BlockSpec((2, _B3, VD), lambda i: (0, i, 0)),
        ],
        out_specs=pl.BlockSpec((_B3, VD), lambda i: (i, 0)),
        out_shape=jax.ShapeDtypeStruct((N, VD), jnp.float32),
    )(num, den)


@jax.jit
def kernel(value, edge_weights, edge_weights_cutoff, edge_index):
    dst = edge_index[1].astype(jnp.int32)
    w = edge_weights_cutoff[:, None] * edge_weights
    wv, pden = _stage1(w, value)
    z = jnp.zeros((128, VD), jnp.float32)
    num = _scatter_pass(_num_body, wv, dst, z, [])
    den = _scatter_pass(_den_body, pden, dst, z,
                        [pltpu.VMEM((128, HD), jnp.float32)])
    return _stage4(num, den)


# speedup vs baseline: 46.5679x; 1.2442x over previous
import jax
import jax.numpy as jnp
import numpy as np
from jax import lax
from jax.experimental import pallas as pl
from jax.experimental.pallas import tpu as pltpu
from jax.experimental.pallas import tpu_sc as plsc

N = 10000
E = 320000
H = 8
VD = 128
HD = VD // H

ROWS = E // 128
GROUP = 4
NGROUPS = ROWS // GROUP
NWORKERS = 32
GSTEPS = -(-NGROUPS // NWORKERS)
NPAD = 10240
ROWS_PER_SUB = NPAD // 16
EPS = 1e-16


_B1 = 2560


def _stage1_body(w_ref, v_ref, wv_ref, pf_ref):
    p = jnp.exp(w_ref[...])
    r = (lax.broadcasted_iota(jnp.int32, (H, VD), 1) // HD
         == lax.broadcasted_iota(jnp.int32, (H, VD), 0)).astype(jnp.float32)
    pfull = jax.lax.dot_general(
        p, r, (((1,), (0,)), ((), ())),
        preferred_element_type=jnp.float32)
    pf_ref[...] = pfull
    wv_ref[...] = v_ref[...] * pfull


def _stage1(w, value):
    grid = (E // _B1,)
    return pl.pallas_call(
        _stage1_body,
        grid=grid,
        in_specs=[
            pl.BlockSpec((_B1, H), lambda i: (i, 0)),
            pl.BlockSpec((_B1, VD), lambda i: (i, 0)),
        ],
        out_specs=[
            pl.BlockSpec((_B1, VD), lambda i: (i, 0)),
            pl.BlockSpec((_B1, VD), lambda i: (i, 0)),
        ],
        out_shape=[
            jax.ShapeDtypeStruct((E, VD), jnp.float32),
            jax.ShapeDtypeStruct((E, VD), jnp.float32),
        ],
    )(w, value)


_sc_mesh = plsc.VectorSubcoreMesh(core_axis_name="c", subcore_axis_name="s")


def _num_body(wv_hbm, dst_hbm, z_hbm, acc_hbm,
              buf, idx0, idx1, idx2, idx3, acc_sh):
    cid = lax.axis_index("c")
    sid = lax.axis_index("s")
    wid = cid * 16 + sid

    row0 = sid * ROWS_PER_SUB
    pltpu.sync_copy(z_hbm, buf)
    for t in range(ROWS_PER_SUB // 128):
        pltpu.sync_copy(buf, acc_sh.at[pl.ds(row0 + t * 128, 128)])
    plsc.subcore_barrier()

    @pl.loop(0, GSTEPS)
    def _(i):
        g = wid + i * NWORKERS

        @pl.when(g < NGROUPS)
        def _():
            for j, idxj in enumerate((idx0, idx1, idx2, idx3)):
                e = (g * GROUP + j) * 128
                pltpu.sync_copy(dst_hbm.at[pl.ds(e, 128)], idxj)
                pltpu.sync_copy(wv_hbm.at[pl.ds(e, 128)], buf)
                pltpu.sync_copy(buf, acc_sh.at[idxj], add=True)

    plsc.subcore_barrier()

    for t in range(ROWS_PER_SUB // 128):
        r = row0 + t * 128
        pltpu.sync_copy(acc_sh.at[pl.ds(r, 128)], buf)
        pltpu.sync_copy(buf, acc_hbm.at[cid, pl.ds(r, 128)])


def _scatter_pass(data, dst_rows, z):
    kern = pl.kernel(
        _num_body,
        out_type=jax.ShapeDtypeStruct((2, NPAD, VD), jnp.float32),
        mesh=_sc_mesh,
        scratch_types=[
            pltpu.VMEM((128, VD), jnp.float32),
            pltpu.VMEM((128,), jnp.int32),
            pltpu.VMEM((128,), jnp.int32),
            pltpu.VMEM((128,), jnp.int32),
            pltpu.VMEM((128,), jnp.int32),
            pltpu.VMEM_SHARED((NPAD, VD), jnp.float32),
        ],
    )
    return kern(data, dst_rows, z)


_B3 = 2000


def _stage4_body(num_ref, den_ref, out_ref):
    nm = num_ref[0] + num_ref[1]
    dn = den_ref[0] + den_ref[1]
    out_ref[...] = nm / (dn + EPS)


def _stage4(num, den):
    grid = (N // _B3,)
    return pl.pallas_call(
        _stage4_body,
        grid=grid,
        in_specs=[
            pl.BlockSpec((2, _B3, VD), lambda i: (0, i, 0)),
            pl.BlockSpec((2, _B3, VD), lambda i: (0, i, 0)),
        ],
        out_specs=pl.BlockSpec((_B3, VD), lambda i: (i, 0)),
        out_shape=jax.ShapeDtypeStruct((N, VD), jnp.float32),
    )(num, den)


@jax.jit
def kernel(value, edge_weights, edge_weights_cutoff, edge_index):
    dst = edge_index[1].astype(jnp.int32)
    w = edge_weights_cutoff[:, None] * edge_weights
    wv, pfull = _stage1(w, value)
    z = jnp.zeros((128, VD), jnp.float32)
    num = _scatter_pass(wv, dst, z)
    den = _scatter_pass(pfull, dst, z)
    return _stage4(num, den)


# speedup vs baseline: 63.0152x; 1.3532x over previous
import jax
import jax.numpy as jnp
import numpy as np
from jax import lax
from jax.experimental import pallas as pl
from jax.experimental.pallas import tpu as pltpu
from jax.experimental.pallas import tpu_sc as plsc

N = 10000
E = 320000
H = 8
VD = 128
HD = VD // H

ROWS = E // 128
GROUP = 4
NGROUPS = ROWS // GROUP
NWORKERS = 32
GSTEPS = -(-NGROUPS // NWORKERS)
NPAD = 10240
ROWS_PER_SUB = NPAD // 16
EPS = 1e-16


_B1 = 2560


def _stage1_body(w_ref, v_ref, wv_ref, pf_ref):
    p = jnp.exp(w_ref[...])
    r = (lax.broadcasted_iota(jnp.int32, (H, VD), 1) // HD
         == lax.broadcasted_iota(jnp.int32, (H, VD), 0)).astype(jnp.float32)
    pfull = jax.lax.dot_general(
        p, r, (((1,), (0,)), ((), ())),
        preferred_element_type=jnp.float32)
    pf_ref[...] = pfull
    wv_ref[...] = v_ref[...] * pfull


def _stage1(w, value):
    grid = (E // _B1,)
    return pl.pallas_call(
        _stage1_body,
        grid=grid,
        in_specs=[
            pl.BlockSpec((_B1, H), lambda i: (i, 0)),
            pl.BlockSpec((_B1, VD), lambda i: (i, 0)),
        ],
        out_specs=[
            pl.BlockSpec((_B1, VD), lambda i: (i, 0)),
            pl.BlockSpec((_B1, VD), lambda i: (i, 0)),
        ],
        out_shape=[
            jax.ShapeDtypeStruct((E, VD), jnp.float32),
            jax.ShapeDtypeStruct((E, VD), jnp.float32),
        ],
    )(w, value)


_sc_mesh = plsc.VectorSubcoreMesh(core_axis_name="c", subcore_axis_name="s")


_NSTEPS = -(-ROWS // NWORKERS)
_NSTEPS2 = _NSTEPS + (_NSTEPS % 2)


def _num_body(wv_hbm, dst_hbm, z_hbm, acc_hbm,
              buf0, buf1, idx0, idx1, bsem0, bsem1, isem0, isem1, acc_sh):
    cid = lax.axis_index("c")
    sid = lax.axis_index("s")
    wid = cid * 16 + sid
    bufs = ((buf0, idx0, bsem0, isem0), (buf1, idx1, bsem1, isem1))

    row0 = sid * ROWS_PER_SUB
    pltpu.sync_copy(z_hbm, buf0)
    for t in range(ROWS_PER_SUB // 128):
        pltpu.sync_copy(buf0, acc_sh.at[pl.ds(row0 + t * 128, 128)])

    for b, (buf, idx, bsem, isem) in enumerate(bufs):
        e = (wid + b * NWORKERS) * 128
        pltpu.async_copy(dst_hbm.at[pl.ds(e, 128)], idx, isem)
        pltpu.async_copy(wv_hbm.at[pl.ds(e, 128)], buf, bsem)

    plsc.subcore_barrier()

    @pl.loop(0, _NSTEPS2, step=2)
    def _(n):
        for b, (buf, idx, bsem, isem) in enumerate(bufs):
            c = wid + (n + b) * NWORKERS

            @pl.when(c < ROWS)
            def _():
                pltpu.make_async_copy(
                    dst_hbm.at[pl.ds(0, 128)], idx, isem).wait()
                pltpu.make_async_copy(
                    wv_hbm.at[pl.ds(0, 128)], buf, bsem).wait()
                pltpu.sync_copy(buf, acc_sh.at[idx], add=True)
                cn = c + 2 * NWORKERS

                @pl.when(cn < ROWS)
                def _():
                    e2 = cn * 128
                    pltpu.async_copy(dst_hbm.at[pl.ds(e2, 128)], idx, isem)
                    pltpu.async_copy(wv_hbm.at[pl.ds(e2, 128)], buf, bsem)

    plsc.subcore_barrier()

    for t in range(ROWS_PER_SUB // 128):
        r = row0 + t * 128
        pltpu.sync_copy(acc_sh.at[pl.ds(r, 128)], buf0)
        pltpu.sync_copy(buf0, acc_hbm.at[cid, pl.ds(r, 128)])


def _scatter_pass(data, dst_rows, z):
    kern = pl.kernel(
        _num_body,
        out_type=jax.ShapeDtypeStruct((2, NPAD, VD), jnp.float32),
        mesh=_sc_mesh,
        scratch_types=[
            pltpu.VMEM((128, VD), jnp.float32),
            pltpu.VMEM((128, VD), jnp.float32),
            pltpu.VMEM((128,), jnp.int32),
            pltpu.VMEM((128,), jnp.int32),
            pltpu.SemaphoreType.DMA,
            pltpu.SemaphoreType.DMA,
            pltpu.SemaphoreType.DMA,
            pltpu.SemaphoreType.DMA,
            pltpu.VMEM_SHARED((NPAD, VD), jnp.float32),
        ],
    )
    return kern(data, dst_rows, z)


_B3 = 2000


def _stage4_body(num_ref, den_ref, out_ref):
    nm = num_ref[0] + num_ref[1]
    dn = den_ref[0] + den_ref[1]
    out_ref[...] = nm / (dn + EPS)


def _stage4(num, den):
    grid = (N // _B3,)
    return pl.pallas_call(
        _stage4_body,
        grid=grid,
        in_specs=[
            pl.BlockSpec((2, _B3, VD), lambda i: (0, i, 0)),
            pl.BlockSpec((2, _B3, VD), lambda i: (0, i, 0)),
        ],
        out_specs=pl.BlockSpec((_B3, VD), lambda i: (i, 0)),
        out_shape=jax.ShapeDtypeStruct((N, VD), jnp.float32),
    )(num, den)


@jax.jit
def kernel(value, edge_weights, edge_weights_cutoff, edge_index):
    dst = edge_index[1].astype(jnp.int32)
    w = edge_weights_cutoff[:, None] * edge_weights
    wv, pfull = _stage1(w, value)
    z = jnp.zeros((128, VD), jnp.float32)
    num = _scatter_pass(wv, dst, z)
    den = _scatter_pass(pfull, dst, z)
    return _stage4(num, den)


# speedup vs baseline: 66.9426x; 1.0623x over previous
import jax
import jax.numpy as jnp
import numpy as np
from jax import lax
from jax.experimental import pallas as pl
from jax.experimental.pallas import tpu as pltpu
from jax.experimental.pallas import tpu_sc as plsc

N = 10000
E = 320000
H = 8
VD = 128
HD = VD // H

ROWS = E // 128
GROUP = 4
NGROUPS = ROWS // GROUP
NWORKERS = 32
GSTEPS = -(-NGROUPS // NWORKERS)
NPAD = 10240
ROWS_PER_SUB = NPAD // 16
EPS = 1e-16


_B1 = 2560


def _stage1_body(pf_ref, v_ref, wv_ref):
    wv_ref[...] = v_ref[...] * pf_ref[...]


def _stage1(pfull, value):
    grid = (E // _B1,)
    return pl.pallas_call(
        _stage1_body,
        grid=grid,
        in_specs=[
            pl.BlockSpec((_B1, VD), lambda i: (i, 0)),
            pl.BlockSpec((_B1, VD), lambda i: (i, 0)),
        ],
        out_specs=pl.BlockSpec((_B1, VD), lambda i: (i, 0)),
        out_shape=jax.ShapeDtypeStruct((E, VD), jnp.float32),
    )(pfull, value)


_sc_mesh = plsc.VectorSubcoreMesh(core_axis_name="c", subcore_axis_name="s")


_NSTEPS = -(-ROWS // NWORKERS)
_NSTEPS2 = _NSTEPS + (_NSTEPS % 2)


def _num_body(wv_hbm, dst_hbm, z_hbm, acc_hbm,
              buf0, buf1, idx0, idx1, bsem0, bsem1, isem0, isem1, acc_sh):
    cid = lax.axis_index("c")
    sid = lax.axis_index("s")
    wid = cid * 16 + sid
    bufs = ((buf0, idx0, bsem0, isem0), (buf1, idx1, bsem1, isem1))

    row0 = sid * ROWS_PER_SUB
    pltpu.sync_copy(z_hbm, buf0)
    for t in range(ROWS_PER_SUB // 128):
        pltpu.sync_copy(buf0, acc_sh.at[pl.ds(row0 + t * 128, 128)])

    for b, (buf, idx, bsem, isem) in enumerate(bufs):
        e = (wid + b * NWORKERS) * 128
        pltpu.async_copy(dst_hbm.at[pl.ds(e, 128)], idx, isem)
        pltpu.async_copy(wv_hbm.at[pl.ds(e, 128)], buf, bsem)

    plsc.subcore_barrier()

    @pl.loop(0, _NSTEPS2, step=2)
    def _(n):
        for b, (buf, idx, bsem, isem) in enumerate(bufs):
            c = wid + (n + b) * NWORKERS

            @pl.when(c < ROWS)
            def _():
                pltpu.make_async_copy(
                    dst_hbm.at[pl.ds(0, 128)], idx, isem).wait()
                pltpu.make_async_copy(
                    wv_hbm.at[pl.ds(0, 128)], buf, bsem).wait()
                pltpu.sync_copy(buf, acc_sh.at[idx], add=True)
                cn = c + 2 * NWORKERS

                @pl.when(cn < ROWS)
                def _():
                    e2 = cn * 128
                    pltpu.async_copy(dst_hbm.at[pl.ds(e2, 128)], idx, isem)
                    pltpu.async_copy(wv_hbm.at[pl.ds(e2, 128)], buf, bsem)

    plsc.subcore_barrier()

    for t in range(ROWS_PER_SUB // 128):
        r = row0 + t * 128
        pltpu.sync_copy(acc_sh.at[pl.ds(r, 128)], buf0)
        pltpu.sync_copy(buf0, acc_hbm.at[cid, pl.ds(r, 128)])


def _scatter_pass(data, dst_rows, z):
    kern = pl.kernel(
        _num_body,
        out_type=jax.ShapeDtypeStruct((2, NPAD, VD), jnp.float32),
        mesh=_sc_mesh,
        scratch_types=[
            pltpu.VMEM((128, VD), jnp.float32),
            pltpu.VMEM((128, VD), jnp.float32),
            pltpu.VMEM((128,), jnp.int32),
            pltpu.VMEM((128,), jnp.int32),
            pltpu.SemaphoreType.DMA,
            pltpu.SemaphoreType.DMA,
            pltpu.SemaphoreType.DMA,
            pltpu.SemaphoreType.DMA,
            pltpu.VMEM_SHARED((NPAD, VD), jnp.float32),
        ],
    )
    return kern(data, dst_rows, z)


_B3 = 2000


def _stage4_body(num_ref, den_ref, out_ref):
    nm = num_ref[0] + num_ref[1]
    dn = den_ref[0] + den_ref[1]
    out_ref[...] = nm / (dn + EPS)


def _stage4(num, den):
    grid = (N // _B3,)
    return pl.pallas_call(
        _stage4_body,
        grid=grid,
        in_specs=[
            pl.BlockSpec((2, _B3, VD), lambda i: (0, i, 0)),
            pl.BlockSpec((2, _B3, VD), lambda i: (0, i, 0)),
        ],
        out_specs=pl.BlockSpec((_B3, VD), lambda i: (i, 0)),
        out_shape=jax.ShapeDtypeStruct((N, VD), jnp.float32),
    )(num, den)


@jax.jit
def kernel(value, edge_weights, edge_weights_cutoff, edge_index):
    dst = edge_index[1].astype(jnp.int32)
    p = jnp.exp(edge_weights_cutoff[:, None] * edge_weights)
    pfull = jnp.reshape(
        jnp.broadcast_to(p[:, :, None], (E, H, HD)), (E, VD))
    wv = _stage1(pfull, value)
    z = jnp.zeros((128, VD), jnp.float32)
    num = _scatter_pass(wv, dst, z)
    den = _scatter_pass(pfull, dst, z)
    return _stage4(num, den)
